# baseline (speedup 1.0000x reference)
import functools

import jax
import jax.numpy as jnp
from jax import lax
from jax.experimental import pallas as pl
from jax.experimental.pallas import tpu as pltpu
from jax.experimental.pallas import tpu_sc as plsc

B = 16384
EMB = 64
HID = 128
NSYMP = 15
Q = 243
NROWS = 1024
NC, NS, L = 2, 16, 16
NW = NC * NS
BPW = B // NW
GROUPS = BPW // L
XBLK = 2048


def _prep_body(x_ref, birth_ref, gender_ref, symp_ref, fused_ref, idx_ref):
    @pl.when(pl.program_id(0) == 0)
    def _build():
        ge = (birth_ref[...] + gender_ref[...]) * (1.0 / 3.0)
        s = symp_ref[...] * (1.0 / 45.0)

        def quint(t):
            T = s[15 * t: 15 * t + 3]
            n = 3
            for i in range(1, 5):
                a = s[15 * t + 3 * i: 15 * t + 3 * i + 3]
                T = (T[:, None, :] + a[None, :, :]).reshape(n * 3, EMB)
                n *= 3
            return T

        t0g = (ge[:, None, :] + quint(0)[None, :, :]).reshape(2 * Q, EMB)
        f = jnp.concatenate(
            [t0g, quint(1), quint(2),
             jnp.zeros((NROWS - 4 * Q, EMB), jnp.float32)], axis=0)
        fused_ref[...] = f.T

    cv = x_ref[:, 1:17]
    m = lax.broadcasted_iota(jnp.int32, (16, 4), 0)
    tcol = lax.broadcasted_iota(jnp.int32, (16, 4), 1)
    mm = jnp.maximum(m - 1, 0)
    tt = mm // 5
    il = mm % 5
    p3 = jnp.where(il == 0, 81.0, jnp.where(il == 1, 27.0,
         jnp.where(il == 2, 9.0, jnp.where(il == 3, 3.0, 1.0))))
    sel = jnp.where((tcol == 0) & (m == 0), 1.0, 0.0)
    sel = sel + jnp.where((m >= 1) & (tcol == tt + 1), p3, 0.0)
    q = jnp.dot(cv, sel, preferred_element_type=jnp.float32,
                precision=lax.Precision.HIGHEST)
    qi = q.astype(jnp.int32)
    p = (qi[:, 3:4] * Q + qi[:, 2:3]) * (2 * Q) + qi[:, 0:1] * Q + qi[:, 1:2]
    idx_ref[...] = p.reshape(XBLK)


def _prep(x, birth, gender, symp):
    return pl.pallas_call(
        _prep_body,
        grid=(B // XBLK,),
        in_specs=[
            pl.BlockSpec((XBLK, 17), lambda i: (i, 0)),
            pl.BlockSpec((1, EMB), lambda i: (0, 0)),
            pl.BlockSpec((2, EMB), lambda i: (0, 0)),
            pl.BlockSpec((NSYMP * 3, EMB), lambda i: (0, 0)),
        ],
        out_specs=[
            pl.BlockSpec((EMB, NROWS), lambda i: (0, 0)),
            pl.BlockSpec((XBLK,), lambda i: (i,)),
        ],
        out_shape=[
            jax.ShapeDtypeStruct((EMB, NROWS), jnp.float32),
            jax.ShapeDtypeStruct((B,), jnp.int32),
        ],
    )(x, birth, gender, symp)


def _sc_body(idx_hbm, fused_hbm, h_hbm, idx_v, fused_v, h_v):
    wid = lax.axis_index("s") * NC + lax.axis_index("c")
    base = wid * BPW
    pltpu.sync_copy(fused_hbm, fused_v)
    pltpu.sync_copy(idx_hbm.at[pl.ds(base, BPW)], idx_v)

    def unpack(p):
        r0 = lax.rem(p, jnp.int32(2 * Q))
        rest = lax.div(p, jnp.int32(2 * Q))
        r1 = lax.rem(rest, jnp.int32(Q)) + 2 * Q
        r2 = lax.div(rest, jnp.int32(Q)) + 3 * Q
        return r0, r1, r2

    def pair(pi, carry):
        off = pi * (2 * L)
        a0, a1, a2 = unpack(idx_v[pl.ds(off, L)])
        b0, b1, b2 = unpack(idx_v[pl.ds(off + L, L)])
        for j in range(EMB):
            fj = jnp.full((L,), j, jnp.int32)
            va = (plsc.load_gather(fused_v, [fj, a0])
                  + plsc.load_gather(fused_v, [fj, a1])
                  + plsc.load_gather(fused_v, [fj, a2]))
            vb = (plsc.load_gather(fused_v, [fj, b0])
                  + plsc.load_gather(fused_v, [fj, b1])
                  + plsc.load_gather(fused_v, [fj, b2]))
            h_v[0, j, pl.ds(off, L)] = va
            h_v[0, j, pl.ds(off + L, L)] = vb
        return carry

    lax.fori_loop(0, GROUPS // 2, pair, 0)
    pltpu.sync_copy(h_v, h_hbm.at[pl.ds(wid, 1)])


@functools.lru_cache(maxsize=1)
def _sc_compute():
    mesh = plsc.VectorSubcoreMesh(core_axis_name="c", subcore_axis_name="s",
                                  num_cores=NC, num_subcores=NS)
    return pl.kernel(
        _sc_body,
        mesh=mesh,
        compiler_params=pltpu.CompilerParams(needs_layout_passes=False),
        out_type=jax.ShapeDtypeStruct((NW, EMB, BPW), jnp.float32),
        scratch_types=[
            pltpu.VMEM((BPW,), jnp.int32),
            pltpu.VMEM((EMB, NROWS), jnp.float32),
            pltpu.VMEM((1, EMB, BPW), jnp.float32),
        ],
    )


MLP_CH = 8


def _mlp_body(ht_ref, w1_ref, b1_ref, w2_ref, b2_ref, out_ref):
    for w in range(MLP_CH):
        ht = ht_ref[w]
        hh = lax.dot_general(w1_ref[...], ht, (((0,), (0,)), ((), ())),
                             preferred_element_type=jnp.float32)
        hh = jnp.maximum(hh + b1_ref[...][:, None], 0.0)
        o = lax.dot_general(w2_ref[...], hh, (((0,), (0,)), ((), ())),
                            preferred_element_type=jnp.float32)
        out_ref[w] = o + b2_ref[...][:, None]


def _mlp(ht, w1, b1, w2, b2):
    return pl.pallas_call(
        _mlp_body,
        grid=(NW // MLP_CH,),
        in_specs=[
            pl.BlockSpec((MLP_CH, EMB, BPW), lambda i: (i, 0, 0)),
            pl.BlockSpec((EMB, HID), lambda i: (0, 0)),
            pl.BlockSpec((HID,), lambda i: (0,)),
            pl.BlockSpec((HID, 1), lambda i: (0, 0)),
            pl.BlockSpec((1,), lambda i: (0,)),
        ],
        out_specs=pl.BlockSpec((MLP_CH, 1, BPW), lambda i: (i, 0, 0)),
        out_shape=jax.ShapeDtypeStruct((NW, 1, BPW), jnp.float32),
    )(ht, w1, b1, w2, b2)


def kernel(x, birth_table, gender_table, symp_tables, W1, b1, W2, b2):
    fused, idxp = _prep(x, birth_table, gender_table,
                        symp_tables.reshape(NSYMP * 3, EMB))
    ht = _sc_compute()(idxp, fused)
    out = _mlp(ht, W1, b1, W2, b2)
    return out.reshape(B, 1)


# speedup vs baseline: 1.0096x; 1.0096x over previous
import functools

import jax
import jax.numpy as jnp
from jax import lax
from jax.experimental import pallas as pl
from jax.experimental.pallas import tpu as pltpu
from jax.experimental.pallas import tpu_sc as plsc

B = 16384
EMB = 64
HID = 128
NSYMP = 15
Q = 243
NROWS = 1024
NC, NS, L = 2, 16, 16
NW = NC * NS
BPW = B // NW
GROUPS = BPW // L


def _fused_body(birth_ref, gender_ref, symp_ref, out_ref):
    ge = (birth_ref[...] + gender_ref[...]) * (1.0 / 3.0)
    s = symp_ref[...] * (1.0 / 45.0)

    def quint(t):
        T = s[15 * t: 15 * t + 3]
        n = 3
        for i in range(1, 5):
            a = s[15 * t + 3 * i: 15 * t + 3 * i + 3]
            T = (T[:, None, :] + a[None, :, :]).reshape(n * 3, EMB)
            n *= 3
        return T

    t0g = (ge[:, None, :] + quint(0)[None, :, :]).reshape(2 * Q, EMB)
    f = jnp.concatenate(
        [t0g, quint(1), quint(2),
         jnp.zeros((NROWS - 4 * Q, EMB), jnp.float32)], axis=0)
    out_ref[...] = f.T


def _build_fused(birth, gender, symp):
    return pl.pallas_call(
        _fused_body,
        out_shape=jax.ShapeDtypeStruct((EMB, NROWS), jnp.float32),
    )(birth, gender, symp)


def _sc_body(x_hbm, fused_hbm, h_hbm, x_v, fused_v, h_v):
    wid = lax.axis_index("s") * NC + lax.axis_index("c")
    base = wid * BPW
    pltpu.sync_copy(fused_hbm, fused_v)
    pltpu.sync_copy(x_hbm.at[pl.ds(base, BPW), :], x_v)

    def indices(off):
        rows = off + lax.iota(jnp.int32, L)

        def xg(col):
            return plsc.load_gather(x_v, [rows, jnp.full((L,), col, jnp.int32)])

        def qv(t):
            acc = xg(2 + 5 * t)
            for i in range(1, 5):
                acc = acc * 3.0 + xg(2 + 5 * t + i)
            return acc

        r0 = (xg(1) * float(Q) + qv(0)).astype(jnp.int32)
        r1 = (qv(1) + float(2 * Q)).astype(jnp.int32)
        r2 = (qv(2) + float(3 * Q)).astype(jnp.int32)
        return r0, r1, r2

    def pair(pi, carry):
        off = pi * (2 * L)
        a0, a1, a2 = indices(off)
        b0, b1, b2 = indices(off + L)
        for j in range(EMB):
            fj = jnp.full((L,), j, jnp.int32)
            va = (plsc.load_gather(fused_v, [fj, a0])
                  + plsc.load_gather(fused_v, [fj, a1])
                  + plsc.load_gather(fused_v, [fj, a2]))
            vb = (plsc.load_gather(fused_v, [fj, b0])
                  + plsc.load_gather(fused_v, [fj, b1])
                  + plsc.load_gather(fused_v, [fj, b2]))
            h_v[0, j, pl.ds(off, L)] = va
            h_v[0, j, pl.ds(off + L, L)] = vb
        return carry

    lax.fori_loop(0, GROUPS // 2, pair, 0)
    pltpu.sync_copy(h_v, h_hbm.at[pl.ds(wid, 1)])


@functools.lru_cache(maxsize=1)
def _sc_compute():
    mesh = plsc.VectorSubcoreMesh(core_axis_name="c", subcore_axis_name="s",
                                  num_cores=NC, num_subcores=NS)
    return pl.kernel(
        _sc_body,
        mesh=mesh,
        compiler_params=pltpu.CompilerParams(needs_layout_passes=False,
                                             use_tc_tiling_on_sc=False),
        out_type=jax.ShapeDtypeStruct((NW, EMB, BPW), jnp.float32),
        scratch_types=[
            pltpu.VMEM((BPW, 17), jnp.float32),
            pltpu.VMEM((EMB, NROWS), jnp.float32),
            pltpu.VMEM((1, EMB, BPW), jnp.float32),
        ],
    )


MLP_CH = 8


def _mlp_body(ht_ref, w1_ref, b1_ref, w2_ref, b2_ref, out_ref):
    for w in range(MLP_CH):
        ht = ht_ref[w]
        hh = lax.dot_general(w1_ref[...], ht, (((0,), (0,)), ((), ())),
                             preferred_element_type=jnp.float32)
        hh = jnp.maximum(hh + b1_ref[...][:, None], 0.0)
        o = lax.dot_general(w2_ref[...], hh, (((0,), (0,)), ((), ())),
                            preferred_element_type=jnp.float32)
        out_ref[w] = o + b2_ref[...][:, None]


def _mlp(ht, w1, b1, w2, b2):
    return pl.pallas_call(
        _mlp_body,
        grid=(NW // MLP_CH,),
        in_specs=[
            pl.BlockSpec((MLP_CH, EMB, BPW), lambda i: (i, 0, 0)),
            pl.BlockSpec((EMB, HID), lambda i: (0, 0)),
            pl.BlockSpec((HID,), lambda i: (0,)),
            pl.BlockSpec((HID, 1), lambda i: (0, 0)),
            pl.BlockSpec((1,), lambda i: (0,)),
        ],
        out_specs=pl.BlockSpec((MLP_CH, 1, BPW), lambda i: (i, 0, 0)),
        out_shape=jax.ShapeDtypeStruct((NW, 1, BPW), jnp.float32),
    )(ht, w1, b1, w2, b2)


def kernel(x, birth_table, gender_table, symp_tables, W1, b1, W2, b2):
    fused = _build_fused(birth_table, gender_table,
                         symp_tables.reshape(NSYMP * 3, EMB))
    ht = _sc_compute()(x, fused)
    out = _mlp(ht, W1, b1, W2, b2)
    return out.reshape(B, 1)


# speedup vs baseline: 1.2323x; 1.2206x over previous
import functools

import jax
import jax.numpy as jnp
from jax import lax
from jax.experimental import pallas as pl
from jax.experimental.pallas import tpu as pltpu
from jax.experimental.pallas import tpu_sc as plsc

B = 16384
EMB = 64
HID = 128
NSYMP = 15
Q = 243
NROWS = 1024
NC, NS, L = 2, 16, 16
NW = NC * NS
BPW = B // NW
GROUPS = BPW // L


def _fused_body(birth_ref, gender_ref, symp_ref, out_ref):
    ge = (birth_ref[...] + gender_ref[...]) * (1.0 / 3.0)
    s = symp_ref[...] * (1.0 / 45.0)

    def quint(t):
        T = s[15 * t: 15 * t + 3]
        n = 3
        for i in range(1, 5):
            a = s[15 * t + 3 * i: 15 * t + 3 * i + 3]
            T = (T[:, None, :] + a[None, :, :]).reshape(n * 3, EMB)
            n *= 3
        return T

    t0g = (ge[:, None, :] + quint(0)[None, :, :]).reshape(2 * Q, EMB)
    f = jnp.concatenate(
        [t0g, quint(1), quint(2),
         jnp.zeros((NROWS - 4 * Q, EMB), jnp.float32)], axis=0)
    out_ref[...] = f.T


def _build_fused(birth, gender, symp):
    return pl.pallas_call(
        _fused_body,
        out_shape=jax.ShapeDtypeStruct((EMB, NROWS), jnp.float32),
    )(birth, gender, symp)


XCH = 128


def _sc_body(x_hbm, fused_hbm, h_hbm, x_v, fused_v, h_v):
    wid = lax.axis_index("s") * NC + lax.axis_index("c")
    base = wid * BPW
    pltpu.sync_copy(fused_hbm, fused_v)

    def indices(off, loff):
        rows = loff + lax.iota(jnp.int32, L)

        def xg(col):
            return plsc.load_gather(x_v, [rows, jnp.full((L,), col, jnp.int32)])

        def qv(t):
            acc = xg(2 + 5 * t)
            for i in range(1, 5):
                acc = acc * 3.0 + xg(2 + 5 * t + i)
            return acc

        r0 = (xg(1) * float(Q) + qv(0)).astype(jnp.int32)
        r1 = (qv(1) + float(2 * Q)).astype(jnp.int32)
        r2 = (qv(2) + float(3 * Q)).astype(jnp.int32)
        return r0, r1, r2

    def pair(pi, carry):
        off = pi * (2 * L)
        loff = lax.rem(off, jnp.int32(XCH))

        @pl.when(loff == 0)
        def _stage():
            pltpu.sync_copy(x_hbm.at[pl.ds(base + off, XCH), :], x_v)

        a0, a1, a2 = indices(off, loff)
        b0, b1, b2 = indices(off + L, loff + L)
        for j in range(EMB):
            fj = jnp.full((L,), j, jnp.int32)
            va = (plsc.load_gather(fused_v, [fj, a0])
                  + plsc.load_gather(fused_v, [fj, a1])
                  + plsc.load_gather(fused_v, [fj, a2]))
            vb = (plsc.load_gather(fused_v, [fj, b0])
                  + plsc.load_gather(fused_v, [fj, b1])
                  + plsc.load_gather(fused_v, [fj, b2]))
            h_v[0, j, pl.ds(off, L)] = va
            h_v[0, j, pl.ds(off + L, L)] = vb
        return carry

    lax.fori_loop(0, GROUPS // 2, pair, 0)
    pltpu.sync_copy(h_v, h_hbm.at[pl.ds(wid, 1)])


@functools.lru_cache(maxsize=1)
def _sc_compute():
    mesh = plsc.VectorSubcoreMesh(core_axis_name="c", subcore_axis_name="s",
                                  num_cores=NC, num_subcores=NS)
    return pl.kernel(
        _sc_body,
        mesh=mesh,
        compiler_params=pltpu.CompilerParams(needs_layout_passes=False),
        out_type=jax.ShapeDtypeStruct((NW, EMB, BPW), jnp.float32),
        scratch_types=[
            pltpu.VMEM((XCH, 17), jnp.float32),
            pltpu.VMEM((EMB, NROWS), jnp.float32),
            pltpu.VMEM((1, EMB, BPW), jnp.float32),
        ],
    )


MLP_CH = 16


def _mlp_body(ht_ref, w1_ref, b1_ref, w2_ref, b2_ref, out_ref):
    for w in range(MLP_CH):
        ht = ht_ref[w]
        hh = lax.dot_general(w1_ref[...], ht, (((0,), (0,)), ((), ())),
                             preferred_element_type=jnp.float32)
        hh = jnp.maximum(hh + b1_ref[...][:, None], 0.0)
        o = lax.dot_general(w2_ref[...], hh, (((0,), (0,)), ((), ())),
                            preferred_element_type=jnp.float32)
        out_ref[w] = o + b2_ref[...][:, None]


def _mlp(ht, w1, b1, w2, b2):
    return pl.pallas_call(
        _mlp_body,
        grid=(NW // MLP_CH,),
        in_specs=[
            pl.BlockSpec((MLP_CH, EMB, BPW), lambda i: (i, 0, 0)),
            pl.BlockSpec((EMB, HID), lambda i: (0, 0)),
            pl.BlockSpec((HID,), lambda i: (0,)),
            pl.BlockSpec((HID, 1), lambda i: (0, 0)),
            pl.BlockSpec((1,), lambda i: (0,)),
        ],
        out_specs=pl.BlockSpec((MLP_CH, 1, BPW), lambda i: (i, 0, 0)),
        out_shape=jax.ShapeDtypeStruct((NW, 1, BPW), jnp.float32),
    )(ht, w1, b1, w2, b2)


def kernel(x, birth_table, gender_table, symp_tables, W1, b1, W2, b2):
    fused = _build_fused(birth_table, gender_table,
                         symp_tables.reshape(NSYMP * 3, EMB))
    ht = _sc_compute()(x, fused)
    out = _mlp(ht, W1, b1, W2, b2)
    return out.reshape(B, 1)


# speedup vs baseline: 1.2344x; 1.0017x over previous
import functools

import jax
import jax.numpy as jnp
from jax import lax
from jax.experimental import pallas as pl
from jax.experimental.pallas import tpu as pltpu
from jax.experimental.pallas import tpu_sc as plsc

B = 16384
EMB = 64
HID = 128
NSYMP = 15
Q = 243
NROWS = 1024
NC, NS, L = 2, 16, 16
NW = NC * NS
BPW = B // NW
GROUPS = BPW // L


def _fused_body(birth_ref, gender_ref, symp_ref, out_ref):
    ge = (birth_ref[...] + gender_ref[...]) * (1.0 / 3.0)
    s = symp_ref[...] * (1.0 / 45.0)

    def quint(t):
        T = s[15 * t: 15 * t + 3]
        n = 3
        for i in range(1, 5):
            a = s[15 * t + 3 * i: 15 * t + 3 * i + 3]
            T = (T[:, None, :] + a[None, :, :]).reshape(n * 3, EMB)
            n *= 3
        return T

    t0g = (ge[:, None, :] + quint(0)[None, :, :]).reshape(2 * Q, EMB)
    f = jnp.concatenate(
        [t0g, quint(1), quint(2),
         jnp.zeros((NROWS - 4 * Q, EMB), jnp.float32)], axis=0)
    out_ref[...] = f.T


def _build_fused(birth, gender, symp):
    return pl.pallas_call(
        _fused_body,
        out_shape=jax.ShapeDtypeStruct((EMB, NROWS), jnp.float32),
    )(birth, gender, symp)


XCH = 128


def _sc_body(x_hbm, fused_hbm, h_hbm, x_v, fused_v, h_v, ridx_v, sem):
    wid = lax.axis_index("s") * NC + lax.axis_index("c")
    base = wid * BPW
    fcp = pltpu.async_copy(fused_hbm, fused_v, sem)

    def indices(loff):
        rows = loff + lax.iota(jnp.int32, L)

        def xg(col):
            return plsc.load_gather(x_v, [rows, jnp.full((L,), col, jnp.int32)])

        def qv(t):
            acc = xg(2 + 5 * t)
            for i in range(1, 5):
                acc = acc * 3.0 + xg(2 + 5 * t + i)
            return acc

        r0 = (xg(1) * float(Q) + qv(0)).astype(jnp.int32)
        r1 = (qv(1) + float(2 * Q)).astype(jnp.int32)
        r2 = (qv(2) + float(3 * Q)).astype(jnp.int32)
        return r0, r1, r2

    for ch in range(BPW // XCH):
        pltpu.sync_copy(x_hbm.at[pl.ds(base + ch * XCH, XCH), :], x_v)
        for gl in range(XCH // L):
            off = ch * XCH + gl * L
            r0, r1, r2 = indices(gl * L)
            ridx_v[0, pl.ds(off, L)] = r0
            ridx_v[1, pl.ds(off, L)] = r1
            ridx_v[2, pl.ds(off, L)] = r2
    fcp.wait()

    def pair(pi, carry):
        off = pi * (2 * L)
        a0 = ridx_v[0, pl.ds(off, L)]
        a1 = ridx_v[1, pl.ds(off, L)]
        a2 = ridx_v[2, pl.ds(off, L)]
        b0 = ridx_v[0, pl.ds(off + L, L)]
        b1 = ridx_v[1, pl.ds(off + L, L)]
        b2 = ridx_v[2, pl.ds(off + L, L)]
        for j in range(EMB):
            fj = jnp.full((L,), j, jnp.int32)
            va = (plsc.load_gather(fused_v, [fj, a0])
                  + plsc.load_gather(fused_v, [fj, a1])
                  + plsc.load_gather(fused_v, [fj, a2]))
            vb = (plsc.load_gather(fused_v, [fj, b0])
                  + plsc.load_gather(fused_v, [fj, b1])
                  + plsc.load_gather(fused_v, [fj, b2]))
            h_v[0, j, pl.ds(off, L)] = va
            h_v[0, j, pl.ds(off + L, L)] = vb
        return carry

    lax.fori_loop(0, GROUPS // 2, pair, 0)
    pltpu.sync_copy(h_v, h_hbm.at[pl.ds(wid, 1)])


@functools.lru_cache(maxsize=1)
def _sc_compute():
    mesh = plsc.VectorSubcoreMesh(core_axis_name="c", subcore_axis_name="s",
                                  num_cores=NC, num_subcores=NS)
    return pl.kernel(
        _sc_body,
        mesh=mesh,
        compiler_params=pltpu.CompilerParams(needs_layout_passes=False),
        out_type=jax.ShapeDtypeStruct((NW, EMB, BPW), jnp.float32),
        scratch_types=[
            pltpu.VMEM((XCH, 17), jnp.float32),
            pltpu.VMEM((EMB, NROWS), jnp.float32),
            pltpu.VMEM((1, EMB, BPW), jnp.float32),
            pltpu.VMEM((3, BPW), jnp.int32),
            pltpu.SemaphoreType.DMA,
        ],
    )


MLP_CH = 16


def _mlp_body(ht_ref, w1_ref, b1_ref, w2_ref, b2_ref, out_ref):
    for w in range(MLP_CH):
        ht = ht_ref[w]
        hh = lax.dot_general(w1_ref[...], ht, (((0,), (0,)), ((), ())),
                             preferred_element_type=jnp.float32)
        hh = jnp.maximum(hh + b1_ref[...][:, None], 0.0)
        o = lax.dot_general(w2_ref[...], hh, (((0,), (0,)), ((), ())),
                            preferred_element_type=jnp.float32)
        out_ref[w] = o + b2_ref[...][:, None]


def _mlp(ht, w1, b1, w2, b2):
    return pl.pallas_call(
        _mlp_body,
        grid=(NW // MLP_CH,),
        in_specs=[
            pl.BlockSpec((MLP_CH, EMB, BPW), lambda i: (i, 0, 0)),
            pl.BlockSpec((EMB, HID), lambda i: (0, 0)),
            pl.BlockSpec((HID,), lambda i: (0,)),
            pl.BlockSpec((HID, 1), lambda i: (0, 0)),
            pl.BlockSpec((1,), lambda i: (0,)),
        ],
        out_specs=pl.BlockSpec((MLP_CH, 1, BPW), lambda i: (i, 0, 0)),
        out_shape=jax.ShapeDtypeStruct((NW, 1, BPW), jnp.float32),
    )(ht, w1, b1, w2, b2)


def kernel(x, birth_table, gender_table, symp_tables, W1, b1, W2, b2):
    fused = _build_fused(birth_table, gender_table,
                         symp_tables.reshape(NSYMP * 3, EMB))
    ht = _sc_compute()(x, fused)
    out = _mlp(ht, W1, b1, W2, b2)
    return out.reshape(B, 1)
